# baseline (speedup 1.0000x reference)
import functools

import jax
import jax.numpy as jnp
from jax import lax
from jax.experimental import pallas as pl
from jax.experimental.pallas import tpu as pltpu
from jax.experimental.pallas import tpu_sc as plsc

N = 10000
E = 320000
D = 128

NUM_CORES = 2
NUM_SUBCORES = 16
ZB = 48
ZMAIN = 624
ZTAIL = N - NUM_SUBCORES * ZMAIN


def _make_spmm(count_per_core: int, base0: int, base1: int, ck: int):
  tile_edges = count_per_core // NUM_SUBCORES
  assert tile_edges * NUM_SUBCORES == count_per_core
  nchunks = tile_edges // ck
  assert nchunks * ck == tile_edges
  assert base0 % 8 == 0 and base1 % 8 == 0 and ck % 8 == 0
  assert nchunks >= 8
  NR = 4
  NI = 8
  n_oct = nchunks // NI
  n_left = nchunks - n_oct * NI

  mesh = plsc.VectorSubcoreMesh(core_axis_name="c", subcore_axis_name="s")

  @functools.partial(
      pl.kernel,
      mesh=mesh,
      out_type=(
          jax.ShapeDtypeStruct((N, D), jnp.float32),
          jax.ShapeDtypeStruct((N, D), jnp.float32),
      ),
      scratch_types=(
          [pltpu.VMEM((ck,), jnp.int32) for _ in range(NI)]
          + [pltpu.VMEM((ck,), jnp.int32) for _ in range(NI)]
          + [pltpu.VMEM((ck, D), jnp.float32) for _ in range(NR)]
          + [pltpu.VMEM((ZB, D), jnp.float32)]
          + [pltpu.VMEM_SHARED((N, D), jnp.float32)]
          + [pltpu.SemaphoreType.DMA for _ in range(2 * NI + 2 * NR)]
      ),
  )
  def spmm(tab0, tab1, src, dst, out0, out1, *scr):
    sidx = scr[0:NI]
    didx = scr[NI:2 * NI]
    rows = scr[2 * NI:2 * NI + NR]
    zbuf = scr[2 * NI + NR]
    acc = scr[2 * NI + NR + 1]
    sems = scr[2 * NI + NR + 2:]
    ssem = sems[0:NI]
    dsem = sems[NI:2 * NI]
    rsem = sems[2 * NI:2 * NI + NR]
    csem = sems[2 * NI + NR:2 * NI + 2 * NR]
    cid = lax.axis_index("c")
    sid = lax.axis_index("s")

    def zrow(i, _):
      def zcol(j, _):
        zbuf[i, pl.ds(j * 16, 16)] = jnp.zeros((16,), jnp.float32)
        return 0
      return lax.fori_loop(0, D // 16, zcol, 0)
    lax.fori_loop(0, ZB, zrow, 0)

    def run(tab, out, edge_base):
      z0 = sid * ZMAIN
      for r in range(ZMAIN // ZB):
        pltpu.sync_copy(zbuf, acc.at[pl.ds(z0 + r * ZB, ZB)])

      @pl.when(sid == NUM_SUBCORES - 1)
      def _():
        pltpu.sync_copy(zbuf.at[pl.ds(0, ZTAIL)],
                        acc.at[pl.ds(NUM_SUBCORES * ZMAIN, ZTAIL)])

      ebase = edge_base + sid * tile_edges

      for c in range(NI):
        off = ebase + c * ck
        pltpu.async_copy(src.at[pl.ds(off, ck)], sidx[c], ssem[c])
        pltpu.async_copy(dst.at[pl.ds(off, ck)], didx[c], dsem[c])
      for c in range(NR - 1):
        pltpu.make_async_copy(src.at[pl.ds(0, ck)], sidx[c], ssem[c]).wait()
        pltpu.async_copy(tab.at[sidx[c]], rows[c], rsem[c])

      plsc.subcore_barrier()

      def chunk_step(j, mr, mi, traced):
        pltpu.make_async_copy(tab.at[sidx[mi]], rows[mr], rsem[mr]).wait()
        pltpu.make_async_copy(dst.at[pl.ds(0, ck)], didx[mi], dsem[mi]).wait()
        pltpu.async_copy(rows[mr], acc.at[didx[mi]], csem[mr], add=True)

        def fire_sidx():
          off = ebase + (j + NI) * ck
          pltpu.async_copy(src.at[pl.ds(off, ck)], sidx[mi], ssem[mi])

        mp = (mr + NR - 1) % NR
        mdp = (mi + NI - 1) % NI

        def wait_prev_scatter():
          pltpu.make_async_copy(rows[mp], acc.at[didx[mi]], csem[mp]).wait()

        def fire_didx():
          off = ebase + (j - 1 + NI) * ck
          pltpu.async_copy(dst.at[pl.ds(off, ck)], didx[mdp], dsem[mdp])

        def fire_gather():
          mg = (mi + NR - 1) % NI
          pltpu.make_async_copy(src.at[pl.ds(0, ck)], sidx[mg], ssem[mg]).wait()
          pltpu.async_copy(tab.at[sidx[mg]], rows[mp], rsem[mp])

        if traced:
          pl.when(j + NI <= nchunks - 1)(fire_sidx)
          pl.when(j >= 1)(wait_prev_scatter)
          pl.when(jnp.logical_and(j >= 1, j - 1 + NI <= nchunks - 1))(fire_didx)
          pl.when(j + NR - 1 <= nchunks - 1)(fire_gather)
        else:
          if j + NI <= nchunks - 1:
            fire_sidx()
          if j >= 1:
            wait_prev_scatter()
          if j >= 1 and j - 1 + NI <= nchunks - 1:
            fire_didx()
          if j + NR - 1 <= nchunks - 1:
            fire_gather()

      def octet(q, _):
        j0 = q * NI
        for m in range(NI):
          chunk_step(j0 + m, m % NR, m, traced=True)
        return 0
      lax.fori_loop(0, n_oct, octet, 0)

      for m in range(n_left):
        c = n_oct * NI + m
        chunk_step(c, c % NR, c % NI, traced=False)

      mlast = (nchunks - 1) % NR
      pltpu.make_async_copy(rows[mlast], acc.at[didx[(nchunks - 1) % NI]],
                            csem[mlast]).wait()

      plsc.subcore_barrier()

      pltpu.sync_copy(acc.at[pl.ds(z0, ZMAIN)], out.at[pl.ds(z0, ZMAIN)])

      @pl.when(sid == NUM_SUBCORES - 1)
      def _():
        t0 = NUM_SUBCORES * ZMAIN
        pltpu.sync_copy(acc.at[pl.ds(t0, ZTAIL)], out.at[pl.ds(t0, ZTAIL)])

    @pl.when(cid == 0)
    def _():
      run(tab0, out0, base0)

    @pl.when(cid == 1)
    def _():
      run(tab1, out1, base1)

  return spmm


CKA = 80
CKB = 80
_spmm_full = _make_spmm(E, 0, 0, CKA)
_spmm_half = _make_spmm(E // 2, 0, E // 2, CKB)


def _mm_body(x_ref, w_ref, b_ref, o0, o1, o2):
  y = jnp.dot(x_ref[...], w_ref[...], preferred_element_type=jnp.float32)
  y = y + b_ref[...]
  o0[...] = y[:, 0:D]
  o1[...] = y[:, D:2 * D]
  o2[...] = y[:, 2 * D:3 * D]


def _asm_body(h0, y1, pa, pb, o):
  o[:, 0:D] = h0[...]
  o[:, D:2 * D] = y1[...]
  o[:, 2 * D:3 * D] = pa[...] + pb[...]


_MB = 2000


def _mm(x, wc, bc):
  return pl.pallas_call(
      _mm_body,
      grid=(N // _MB,),
      in_specs=[
          pl.BlockSpec((_MB, D), lambda i: (i, 0)),
          pl.BlockSpec((D, 3 * D), lambda i: (0, 0)),
          pl.BlockSpec((1, 3 * D), lambda i: (0, 0)),
      ],
      out_specs=[
          pl.BlockSpec((_MB, D), lambda i: (i, 0)),
          pl.BlockSpec((_MB, D), lambda i: (i, 0)),
          pl.BlockSpec((_MB, D), lambda i: (i, 0)),
      ],
      out_shape=[jax.ShapeDtypeStruct((N, D), jnp.float32)] * 3,
  )(x, wc, bc)


def _asm(h0, y1, pa, pb):
  return pl.pallas_call(
      _asm_body,
      grid=(N // _MB,),
      in_specs=[pl.BlockSpec((_MB, D), lambda i: (i, 0))] * 4,
      out_specs=pl.BlockSpec((_MB, 3 * D), lambda i: (i, 0)),
      out_shape=jax.ShapeDtypeStruct((N, 3 * D), jnp.float32),
  )(h0, y1, pa, pb)


def kernel(x, edge_index, W0, b0, W1, b1, W2, b2):
  wc = jnp.concatenate([W0.T, W1.T, W2.T], axis=1)
  bc = jnp.concatenate([b0, b1, b2]).reshape(1, 3 * D)
  src = edge_index[0]
  dst = edge_index[1]
  h0, p1, p2 = _mm(x, wc, bc)
  y1, t = _spmm_full(p1, p2, src, dst)
  pa, pb = _spmm_half(t, t, src, dst)
  return _asm(h0, y1, pa, pb)


# speedup vs baseline: 1.0853x; 1.0853x over previous
import functools

import jax
import jax.numpy as jnp
from jax import lax
from jax.experimental import pallas as pl
from jax.experimental.pallas import tpu as pltpu
from jax.experimental.pallas import tpu_sc as plsc

N = 10000
E = 320000
D = 128

NUM_CORES = 2
NUM_SUBCORES = 16
ZB = 48
ZMAIN = 624
ZTAIL = N - NUM_SUBCORES * ZMAIN


def _make_spmm(count_per_core: int, base0: int, base1: int, ck: int):
  tile_edges = count_per_core // NUM_SUBCORES
  assert tile_edges * NUM_SUBCORES == count_per_core
  nchunks = tile_edges // ck
  assert nchunks * ck == tile_edges
  assert base0 % 8 == 0 and base1 % 8 == 0 and ck % 8 == 0
  assert nchunks >= 8
  NR = 4
  NI = 8
  n_oct = nchunks // NI
  n_left = nchunks - n_oct * NI

  mesh = plsc.VectorSubcoreMesh(core_axis_name="c", subcore_axis_name="s")

  @functools.partial(
      pl.kernel,
      mesh=mesh,
      out_type=(
          jax.ShapeDtypeStruct((N, D), jnp.float32),
          jax.ShapeDtypeStruct((N, D), jnp.float32),
      ),
      scratch_types=(
          [pltpu.VMEM((ck,), jnp.int32) for _ in range(NI)]
          + [pltpu.VMEM((ck,), jnp.int32) for _ in range(NI)]
          + [pltpu.VMEM((ck, D), jnp.float32) for _ in range(NR)]
          + [pltpu.VMEM((ZB, D), jnp.float32)]
          + [pltpu.VMEM_SHARED((N, D), jnp.float32)]
          + [pltpu.SemaphoreType.DMA for _ in range(2 * NI + NR)]
      ),
  )
  def spmm(tab0, tab1, ei, out0, out1, *scr):
    sidx = scr[0:NI]
    didx = scr[NI:2 * NI]
    rows = scr[2 * NI:2 * NI + NR]
    zbuf = scr[2 * NI + NR]
    acc = scr[2 * NI + NR + 1]
    sems = scr[2 * NI + NR + 2:]
    ssem = sems[0:NI]
    dsem = sems[NI:2 * NI]
    rsem = sems[2 * NI:2 * NI + NR]
    cid = lax.axis_index("c")
    sid = lax.axis_index("s")

    def zrow(i, _):
      def zcol(j, _):
        zbuf[i, pl.ds(j * 16, 16)] = jnp.zeros((16,), jnp.float32)
        return 0
      return lax.fori_loop(0, D // 16, zcol, 0)
    lax.fori_loop(0, ZB, zrow, 0)

    def run(tab, out, edge_base):
      z0 = sid * ZMAIN
      for r in range(ZMAIN // ZB):
        pltpu.sync_copy(zbuf, acc.at[pl.ds(z0 + r * ZB, ZB)])

      @pl.when(sid == NUM_SUBCORES - 1)
      def _():
        pltpu.sync_copy(zbuf.at[pl.ds(0, ZTAIL)],
                        acc.at[pl.ds(NUM_SUBCORES * ZMAIN, ZTAIL)])

      ebase = edge_base + sid * tile_edges

      for c in range(NI):
        off = ebase + c * ck
        pltpu.async_copy(ei.at[pl.ds(off, ck)], sidx[c], ssem[c])
        pltpu.async_copy(ei.at[pl.ds(E + off, ck)], didx[c], dsem[c])
      for c in range(NR - 1):
        pltpu.make_async_copy(ei.at[pl.ds(0, ck)], sidx[c], ssem[c]).wait()
        pltpu.async_copy(tab.at[sidx[c]], rows[c], rsem[c])

      plsc.subcore_barrier()

      def chunk_step(j, mr, mi, traced):
        pltpu.make_async_copy(tab.at[sidx[mi]], rows[mr], rsem[mr]).wait()
        pltpu.make_async_copy(ei.at[pl.ds(0, ck)], didx[mi], dsem[mi]).wait()
        pltpu.sync_copy(rows[mr], acc.at[didx[mi]], add=True)

        def fire_idx():
          off = ebase + (j + NI) * ck
          pltpu.async_copy(ei.at[pl.ds(off, ck)], sidx[mi], ssem[mi])
          pltpu.async_copy(ei.at[pl.ds(E + off, ck)], didx[mi], dsem[mi])

        def fire_gather():
          mg = (mi + NR - 1) % NI
          pltpu.make_async_copy(ei.at[pl.ds(0, ck)], sidx[mg], ssem[mg]).wait()
          pltpu.async_copy(tab.at[sidx[mg]], rows[(mr + NR - 1) % NR],
                           rsem[(mr + NR - 1) % NR])

        if traced:
          pl.when(j + NI <= nchunks - 1)(fire_idx)
          pl.when(j + NR - 1 <= nchunks - 1)(fire_gather)
        else:
          if j + NI <= nchunks - 1:
            fire_idx()
          if j + NR - 1 <= nchunks - 1:
            fire_gather()

      def octet(q, _):
        j0 = q * NI
        for m in range(NI):
          chunk_step(j0 + m, m % NR, m, traced=True)
        return 0
      lax.fori_loop(0, n_oct, octet, 0)

      for m in range(n_left):
        c = n_oct * NI + m
        chunk_step(c, c % NR, c % NI, traced=False)

      plsc.subcore_barrier()

      pltpu.sync_copy(acc.at[pl.ds(z0, ZMAIN)], out.at[pl.ds(z0, ZMAIN)])

      @pl.when(sid == NUM_SUBCORES - 1)
      def _():
        t0 = NUM_SUBCORES * ZMAIN
        pltpu.sync_copy(acc.at[pl.ds(t0, ZTAIL)], out.at[pl.ds(t0, ZTAIL)])

    @pl.when(cid == 0)
    def _():
      run(tab0, out0, base0)

    @pl.when(cid == 1)
    def _():
      run(tab1, out1, base1)

  return spmm


CKA = 80
CKB = 80
_spmm_full = _make_spmm(E, 0, 0, CKA)
_spmm_half = _make_spmm(E // 2, 0, E // 2, CKB)


def _mm_body(x_ref, w_ref, b_ref, o0, o1, o2):
  y = jnp.dot(x_ref[...], w_ref[...], preferred_element_type=jnp.float32)
  y = y + b_ref[...]
  o0[...] = y[:, 0:D]
  o1[...] = y[:, D:2 * D]
  o2[...] = y[:, 2 * D:3 * D]


def _asm_body(h0, y1, pa, pb, o):
  o[:, 0:D] = h0[...]
  o[:, D:2 * D] = y1[...]
  o[:, 2 * D:3 * D] = pa[...] + pb[...]


_MB = 2000


def _mm(x, wc, bc):
  return pl.pallas_call(
      _mm_body,
      grid=(N // _MB,),
      in_specs=[
          pl.BlockSpec((_MB, D), lambda i: (i, 0)),
          pl.BlockSpec((D, 3 * D), lambda i: (0, 0)),
          pl.BlockSpec((1, 3 * D), lambda i: (0, 0)),
      ],
      out_specs=[
          pl.BlockSpec((_MB, D), lambda i: (i, 0)),
          pl.BlockSpec((_MB, D), lambda i: (i, 0)),
          pl.BlockSpec((_MB, D), lambda i: (i, 0)),
      ],
      out_shape=[jax.ShapeDtypeStruct((N, D), jnp.float32)] * 3,
  )(x, wc, bc)


def _asm(h0, y1, pa, pb):
  return pl.pallas_call(
      _asm_body,
      grid=(N // _MB,),
      in_specs=[pl.BlockSpec((_MB, D), lambda i: (i, 0))] * 4,
      out_specs=pl.BlockSpec((_MB, 3 * D), lambda i: (i, 0)),
      out_shape=jax.ShapeDtypeStruct((N, 3 * D), jnp.float32),
  )(h0, y1, pa, pb)


def kernel(x, edge_index, W0, b0, W1, b1, W2, b2):
  wc = jnp.concatenate([W0.T, W1.T, W2.T], axis=1)
  bc = jnp.concatenate([b0, b1, b2]).reshape(1, 3 * D)
  ei_flat = edge_index.reshape(2 * E)
  h0, p1, p2 = _mm(x, wc, bc)
  y1, t = _spmm_full(p1, p2, ei_flat)
  pa, pb = _spmm_half(t, t, ei_flat)
  return _asm(h0, y1, pa, pb)


# speedup vs baseline: 1.0934x; 1.0074x over previous
import functools

import jax
import jax.numpy as jnp
from jax import lax
from jax.experimental import pallas as pl
from jax.experimental.pallas import tpu as pltpu
from jax.experimental.pallas import tpu_sc as plsc

N = 10000
E = 320000
D = 128

NUM_CORES = 2
NUM_SUBCORES = 16
ZB = 48
ZMAIN = 624
ZTAIL = N - NUM_SUBCORES * ZMAIN


def _make_spmm(count_per_core: int, base0: int, base1: int, ck: int):
  tile_edges = count_per_core // NUM_SUBCORES
  assert tile_edges * NUM_SUBCORES == count_per_core
  nchunks = tile_edges // ck
  assert nchunks * ck == tile_edges
  assert base0 % 8 == 0 and base1 % 8 == 0 and ck % 8 == 0
  assert nchunks >= 8
  NR = 4
  NI = 8
  n_oct = nchunks // NI
  n_left = nchunks - n_oct * NI

  mesh = plsc.VectorSubcoreMesh(core_axis_name="c", subcore_axis_name="s")

  @functools.partial(
      pl.kernel,
      mesh=mesh,
      out_type=(
          jax.ShapeDtypeStruct((N, D), jnp.float32),
          jax.ShapeDtypeStruct((N, D), jnp.float32),
      ),
      scratch_types=(
          [pltpu.VMEM((ck,), jnp.int32) for _ in range(NI)]
          + [pltpu.VMEM((ck,), jnp.int32) for _ in range(NI)]
          + [pltpu.VMEM((ck, D), jnp.float32) for _ in range(NR)]
          + [pltpu.VMEM((ZB, D), jnp.float32)]
          + [pltpu.VMEM_SHARED((N, D), jnp.float32)]
          + [pltpu.SemaphoreType.DMA for _ in range(2 * NI + NR)]
      ),
  )
  def spmm(tab0, tab1, ei, out0, out1, *scr):
    sidx = scr[0:NI]
    didx = scr[NI:2 * NI]
    rows = scr[2 * NI:2 * NI + NR]
    zbuf = scr[2 * NI + NR]
    acc = scr[2 * NI + NR + 1]
    sems = scr[2 * NI + NR + 2:]
    ssem = sems[0:NI]
    dsem = sems[NI:2 * NI]
    rsem = sems[2 * NI:2 * NI + NR]
    cid = lax.axis_index("c")
    sid = lax.axis_index("s")

    def zrow(i, _):
      def zcol(j, _):
        zbuf[i, pl.ds(j * 16, 16)] = jnp.zeros((16,), jnp.float32)
        return 0
      return lax.fori_loop(0, D // 16, zcol, 0)
    lax.fori_loop(0, ZB, zrow, 0)

    def run(tab, out, edge_base):
      z0 = sid * ZMAIN
      for r in range(ZMAIN // ZB):
        pltpu.sync_copy(zbuf, acc.at[pl.ds(z0 + r * ZB, ZB)])

      @pl.when(sid == NUM_SUBCORES - 1)
      def _():
        pltpu.sync_copy(zbuf.at[pl.ds(0, ZTAIL)],
                        acc.at[pl.ds(NUM_SUBCORES * ZMAIN, ZTAIL)])

      ebase = edge_base + sid * tile_edges

      for c in range(NI):
        off = ebase + c * ck
        pltpu.async_copy(ei.at[pl.ds(off, ck)], sidx[c], ssem[c])
        pltpu.async_copy(ei.at[pl.ds(E + off, ck)], didx[c], dsem[c])
      for c in range(NR - 1):
        pltpu.make_async_copy(ei.at[pl.ds(0, ck)], sidx[c], ssem[c]).wait()
        pltpu.async_copy(tab.at[sidx[c]], rows[c], rsem[c])

      plsc.subcore_barrier()

      def chunk_step(j, mr, mi, traced):
        pltpu.make_async_copy(tab.at[sidx[mi]], rows[mr], rsem[mr]).wait()
        pltpu.make_async_copy(ei.at[pl.ds(0, ck)], didx[mi], dsem[mi]).wait()
        pltpu.sync_copy(rows[mr], acc.at[didx[mi]], add=True)

        def fire_idx():
          off = ebase + (j + NI) * ck
          pltpu.async_copy(ei.at[pl.ds(off, ck)], sidx[mi], ssem[mi])
          pltpu.async_copy(ei.at[pl.ds(E + off, ck)], didx[mi], dsem[mi])

        def fire_gather():
          mg = (mi + NR - 1) % NI
          pltpu.make_async_copy(ei.at[pl.ds(0, ck)], sidx[mg], ssem[mg]).wait()
          pltpu.async_copy(tab.at[sidx[mg]], rows[(mr + NR - 1) % NR],
                           rsem[(mr + NR - 1) % NR])

        if traced:
          pl.when(j + NI <= nchunks - 1)(fire_idx)
          pl.when(j + NR - 1 <= nchunks - 1)(fire_gather)
        else:
          if j + NI <= nchunks - 1:
            fire_idx()
          if j + NR - 1 <= nchunks - 1:
            fire_gather()

      def octet(q, _):
        j0 = q * NI
        for m in range(NI):
          chunk_step(j0 + m, m % NR, m, traced=True)
        return 0
      lax.fori_loop(0, n_oct, octet, 0)

      for m in range(n_left):
        c = n_oct * NI + m
        chunk_step(c, c % NR, c % NI, traced=False)

      plsc.subcore_barrier()

      pltpu.sync_copy(acc.at[pl.ds(z0, ZMAIN)], out.at[pl.ds(z0, ZMAIN)])

      @pl.when(sid == NUM_SUBCORES - 1)
      def _():
        t0 = NUM_SUBCORES * ZMAIN
        pltpu.sync_copy(acc.at[pl.ds(t0, ZTAIL)], out.at[pl.ds(t0, ZTAIL)])

    @pl.when(cid == 0)
    def _():
      run(tab0, out0, base0)

    @pl.when(cid == 1)
    def _():
      run(tab1, out1, base1)

  return spmm


CKA = 80
CKB = 80
_spmm_full = _make_spmm(E, 0, 0, CKA)
_spmm_half = _make_spmm(E // 2, 0, E // 2, CKB)


def _mm_body(x_ref, w_ref, b_ref, o1, o2):
  y = jnp.dot(x_ref[...], w_ref[...], preferred_element_type=jnp.float32)
  y = y + b_ref[...]
  o1[...] = y[:, 0:D]
  o2[...] = y[:, D:2 * D]


def _mmh_body(x_ref, w_ref, b_ref, o0):
  o0[...] = jnp.dot(x_ref[...], w_ref[...],
                    preferred_element_type=jnp.float32) + b_ref[...]


def _asm_body(h0, y1, pa, pb, o):
  o[:, 0:D] = h0[...]
  o[:, D:2 * D] = y1[...]
  o[:, 2 * D:3 * D] = pa[...] + pb[...]


_MB = 2000


def _mm(x, wc, bc):
  return pl.pallas_call(
      _mm_body,
      grid=(N // _MB,),
      in_specs=[
          pl.BlockSpec((_MB, D), lambda i: (i, 0)),
          pl.BlockSpec((D, 2 * D), lambda i: (0, 0)),
          pl.BlockSpec((1, 2 * D), lambda i: (0, 0)),
      ],
      out_specs=[
          pl.BlockSpec((_MB, D), lambda i: (i, 0)),
          pl.BlockSpec((_MB, D), lambda i: (i, 0)),
      ],
      out_shape=[jax.ShapeDtypeStruct((N, D), jnp.float32)] * 2,
  )(x, wc, bc)


def _mmh(x, w0t, b0):
  return pl.pallas_call(
      _mmh_body,
      grid=(N // _MB,),
      in_specs=[
          pl.BlockSpec((_MB, D), lambda i: (i, 0)),
          pl.BlockSpec((D, D), lambda i: (0, 0)),
          pl.BlockSpec((1, D), lambda i: (0, 0)),
      ],
      out_specs=pl.BlockSpec((_MB, D), lambda i: (i, 0)),
      out_shape=jax.ShapeDtypeStruct((N, D), jnp.float32),
  )(x, w0t, b0)


def _asm(h0, y1, pa, pb):
  return pl.pallas_call(
      _asm_body,
      grid=(N // _MB,),
      in_specs=[pl.BlockSpec((_MB, D), lambda i: (i, 0))] * 4,
      out_specs=pl.BlockSpec((_MB, 3 * D), lambda i: (i, 0)),
      out_shape=jax.ShapeDtypeStruct((N, 3 * D), jnp.float32),
  )(h0, y1, pa, pb)


def kernel(x, edge_index, W0, b0, W1, b1, W2, b2):
  wc = jnp.concatenate([W1.T, W2.T], axis=1)
  bc = jnp.concatenate([b1, b2]).reshape(1, 2 * D)
  ei_flat = edge_index.reshape(2 * E)
  p1, p2 = _mm(x, wc, bc)
  y1, t = _spmm_full(p1, p2, ei_flat)
  h0 = _mmh(x, W0.T, b0.reshape(1, D))
  pa, pb = _spmm_half(t, t, ei_flat)
  return _asm(h0, y1, pa, pb)


# speedup vs baseline: 1.0965x; 1.0028x over previous
import functools

import jax
import jax.numpy as jnp
from jax import lax
from jax.experimental import pallas as pl
from jax.experimental.pallas import tpu as pltpu
from jax.experimental.pallas import tpu_sc as plsc

N = 10000
E = 320000
D = 128

NUM_CORES = 2
NUM_SUBCORES = 16
ZB = 48
ZMAIN = 624
ZTAIL = N - NUM_SUBCORES * ZMAIN


def _make_spmm(count_per_core: int, base0: int, base1: int, ck: int):
  tile_edges = count_per_core // NUM_SUBCORES
  assert tile_edges * NUM_SUBCORES == count_per_core
  nchunks = tile_edges // ck
  assert nchunks * ck == tile_edges
  assert base0 % 8 == 0 and base1 % 8 == 0 and ck % 8 == 0
  assert nchunks >= 8
  NR = 4
  NI = 8
  n_oct = nchunks // NI
  n_left = nchunks - n_oct * NI

  mesh = plsc.VectorSubcoreMesh(core_axis_name="c", subcore_axis_name="s")

  @functools.partial(
      pl.kernel,
      mesh=mesh,
      out_type=(
          jax.ShapeDtypeStruct((N, D), jnp.float32),
          jax.ShapeDtypeStruct((N, D), jnp.float32),
      ),
      scratch_types=(
          [pltpu.VMEM((ck,), jnp.int32) for _ in range(NI)]
          + [pltpu.VMEM((ck,), jnp.int32) for _ in range(NI)]
          + [pltpu.VMEM((ck, D), jnp.float32) for _ in range(NR)]
          + [pltpu.VMEM((ZB, D), jnp.float32)]
          + [pltpu.VMEM_SHARED((N, D), jnp.float32)]
          + [pltpu.SemaphoreType.DMA for _ in range(2 * NI + NR)]
      ),
  )
  def spmm(tab0, tab1, ei, out0, out1, *scr):
    sidx = scr[0:NI]
    didx = scr[NI:2 * NI]
    rows = scr[2 * NI:2 * NI + NR]
    zbuf = scr[2 * NI + NR]
    acc = scr[2 * NI + NR + 1]
    sems = scr[2 * NI + NR + 2:]
    ssem = sems[0:NI]
    dsem = sems[NI:2 * NI]
    rsem = sems[2 * NI:2 * NI + NR]
    cid = lax.axis_index("c")
    sid = lax.axis_index("s")

    def zrow(i, _):
      def zcol(j, _):
        zbuf[i, pl.ds(j * 16, 16)] = jnp.zeros((16,), jnp.float32)
        return 0
      return lax.fori_loop(0, D // 16, zcol, 0)
    lax.fori_loop(0, ZB, zrow, 0)

    def run(tab, out, edge_base):
      ebase = edge_base + sid * tile_edges

      for c in range(NI):
        off = ebase + c * ck
        pltpu.async_copy(ei.at[pl.ds(off, ck)], sidx[c], ssem[c])
        pltpu.async_copy(ei.at[pl.ds(E + off, ck)], didx[c], dsem[c])

      z0 = sid * ZMAIN
      for r in range(ZMAIN // ZB):
        pltpu.sync_copy(zbuf, acc.at[pl.ds(z0 + r * ZB, ZB)])

      @pl.when(sid == NUM_SUBCORES - 1)
      def _():
        pltpu.sync_copy(zbuf.at[pl.ds(0, ZTAIL)],
                        acc.at[pl.ds(NUM_SUBCORES * ZMAIN, ZTAIL)])

      for c in range(NR - 1):
        pltpu.make_async_copy(ei.at[pl.ds(0, ck)], sidx[c], ssem[c]).wait()
        pltpu.async_copy(tab.at[sidx[c]], rows[c], rsem[c])

      plsc.subcore_barrier()

      def chunk_step(j, mr, mi, traced):
        pltpu.make_async_copy(tab.at[sidx[mi]], rows[mr], rsem[mr]).wait()
        pltpu.make_async_copy(ei.at[pl.ds(0, ck)], didx[mi], dsem[mi]).wait()
        pltpu.sync_copy(rows[mr], acc.at[didx[mi]], add=True)

        def fire_idx():
          off = ebase + (j + NI) * ck
          pltpu.async_copy(ei.at[pl.ds(off, ck)], sidx[mi], ssem[mi])
          pltpu.async_copy(ei.at[pl.ds(E + off, ck)], didx[mi], dsem[mi])

        def fire_gather():
          mg = (mi + NR - 1) % NI
          pltpu.make_async_copy(ei.at[pl.ds(0, ck)], sidx[mg], ssem[mg]).wait()
          pltpu.async_copy(tab.at[sidx[mg]], rows[(mr + NR - 1) % NR],
                           rsem[(mr + NR - 1) % NR])

        if traced:
          pl.when(j + NI <= nchunks - 1)(fire_idx)
          pl.when(j + NR - 1 <= nchunks - 1)(fire_gather)
        else:
          if j + NI <= nchunks - 1:
            fire_idx()
          if j + NR - 1 <= nchunks - 1:
            fire_gather()

      def octet(q, _):
        j0 = q * NI
        for m in range(NI):
          chunk_step(j0 + m, m % NR, m, traced=True)
        return 0
      lax.fori_loop(0, n_oct, octet, 0)

      for m in range(n_left):
        c = n_oct * NI + m
        chunk_step(c, c % NR, c % NI, traced=False)

      plsc.subcore_barrier()

      pltpu.sync_copy(acc.at[pl.ds(z0, ZMAIN)], out.at[pl.ds(z0, ZMAIN)])

      @pl.when(sid == NUM_SUBCORES - 1)
      def _():
        t0 = NUM_SUBCORES * ZMAIN
        pltpu.sync_copy(acc.at[pl.ds(t0, ZTAIL)], out.at[pl.ds(t0, ZTAIL)])

    @pl.when(cid == 0)
    def _():
      run(tab0, out0, base0)

    @pl.when(cid == 1)
    def _():
      run(tab1, out1, base1)

  return spmm


CKA = 80
CKB = 80
_spmm_full = _make_spmm(E, 0, 0, CKA)
_spmm_half = _make_spmm(E // 2, 0, E // 2, CKB)


def _mm_body(x_ref, w_ref, b_ref, o1, o2):
  y = jnp.dot(x_ref[...], w_ref[...], preferred_element_type=jnp.float32)
  y = y + b_ref[...]
  o1[...] = y[:, 0:D]
  o2[...] = y[:, D:2 * D]


def _mmh_body(x_ref, w_ref, b_ref, o0):
  o0[...] = jnp.dot(x_ref[...], w_ref[...],
                    preferred_element_type=jnp.float32) + b_ref[...]


def _asm_body(h0, y1, pa, pb, o):
  o[:, 0:D] = h0[...]
  o[:, D:2 * D] = y1[...]
  o[:, 2 * D:3 * D] = pa[...] + pb[...]


_MB = 2000


def _mm(x, wc, bc):
  return pl.pallas_call(
      _mm_body,
      grid=(N // _MB,),
      in_specs=[
          pl.BlockSpec((_MB, D), lambda i: (i, 0)),
          pl.BlockSpec((D, 2 * D), lambda i: (0, 0)),
          pl.BlockSpec((1, 2 * D), lambda i: (0, 0)),
      ],
      out_specs=[
          pl.BlockSpec((_MB, D), lambda i: (i, 0)),
          pl.BlockSpec((_MB, D), lambda i: (i, 0)),
      ],
      out_shape=[jax.ShapeDtypeStruct((N, D), jnp.float32)] * 2,
  )(x, wc, bc)


def _mmh(x, w0t, b0):
  return pl.pallas_call(
      _mmh_body,
      grid=(N // _MB,),
      in_specs=[
          pl.BlockSpec((_MB, D), lambda i: (i, 0)),
          pl.BlockSpec((D, D), lambda i: (0, 0)),
          pl.BlockSpec((1, D), lambda i: (0, 0)),
      ],
      out_specs=pl.BlockSpec((_MB, D), lambda i: (i, 0)),
      out_shape=jax.ShapeDtypeStruct((N, D), jnp.float32),
  )(x, w0t, b0)


def _asm(h0, y1, pa, pb):
  return pl.pallas_call(
      _asm_body,
      grid=(N // _MB,),
      in_specs=[pl.BlockSpec((_MB, D), lambda i: (i, 0))] * 4,
      out_specs=pl.BlockSpec((_MB, 3 * D), lambda i: (i, 0)),
      out_shape=jax.ShapeDtypeStruct((N, 3 * D), jnp.float32),
  )(h0, y1, pa, pb)


def kernel(x, edge_index, W0, b0, W1, b1, W2, b2):
  wc = jnp.concatenate([W1.T, W2.T], axis=1)
  bc = jnp.concatenate([b1, b2]).reshape(1, 2 * D)
  ei_flat = edge_index.reshape(2 * E)
  p1, p2 = _mm(x, wc, bc)
  y1, t = _spmm_full(p1, p2, ei_flat)
  h0 = _mmh(x, W0.T, b0.reshape(1, D))
  pa, pb = _spmm_half(t, t, ei_flat)
  return _asm(h0, y1, pa, pb)
